# Initial kernel scaffold; baseline (speedup 1.0000x reference)
import jax
import jax.numpy as jnp
from jax.experimental import pallas as pl


def kernel(fingerprints, atom_degree_list, bond_feature, bond_degree_list, i_bond_j, adjacency, words, embed_table, W_bond, b_bond, W_nfc, b_nfc, W_sfc, b_sfc, W_sub, b_sub, W_fc, b_fc, W_out, b_out, W_int, b_int):
    raise NotImplementedError("write your pallas kernel here")


# speedup vs baseline: 2.6178x; 2.6178x over previous
import functools

import jax
import jax.numpy as jnp
from jax import lax
from jax.experimental import pallas as pl
from jax.experimental.pallas import tpu as pltpu
from jax.experimental.pallas import tpu_sc as plsc

N = 10000
NPAD = 10240
DEG = 16
NB = 160000
DIM = 128
NW = 32


def _mesh():
    return plsc.VectorSubcoreMesh(
        core_axis_name="c", subcore_axis_name="s", num_cores=2, num_subcores=16
    )


def _wid():
    return lax.axis_index("s") * 2 + lax.axis_index("c")


def _linear(x, wt, b=None, act=None, R=640):
    M, K = x.shape
    Do = wt.shape[1]
    in_specs = [
        pl.BlockSpec((R, K), lambda i: (i, 0)),
        pl.BlockSpec((K, Do), lambda i: (0, 0)),
    ]
    args = [x, wt]
    if b is not None:
        in_specs.append(pl.BlockSpec((1, Do), lambda i: (0, 0)))
        args.append(b.reshape(1, Do))

    def body(*refs):
        x_ref, w_ref = refs[0], refs[1]
        o_ref = refs[-1]
        y = jnp.dot(x_ref[...], w_ref[...], preferred_element_type=jnp.float32)
        if b is not None:
            y = y + refs[2][...]
        if act == "relu":
            y = jnp.maximum(y, 0.0)
        o_ref[...] = y

    return pl.pallas_call(
        body,
        grid=(M // R,),
        in_specs=in_specs,
        out_specs=pl.BlockSpec((R, Do), lambda i: (i, 0)),
        out_shape=jax.ShapeDtypeStruct((M, Do), jnp.float32),
    )(*args)


def _adj_step(adjacency, hs, xs, BM=400):
    def body(a_ref, h_ref, x_ref, o_ref):
        o_ref[...] = x_ref[...] + jnp.dot(
            a_ref[...], h_ref[...], preferred_element_type=jnp.float32
        )

    return pl.pallas_call(
        body,
        grid=(N // BM,),
        in_specs=[
            pl.BlockSpec((BM, N), lambda i: (i, 0)),
            pl.BlockSpec((N, DIM), lambda i: (0, 0)),
            pl.BlockSpec((BM, DIM), lambda i: (i, 0)),
        ],
        out_specs=pl.BlockSpec((BM, DIM), lambda i: (i, 0)),
        out_shape=jax.ShapeDtypeStruct((N, DIM), jnp.float32),
    )(adjacency, hs, xs)


def _epilogue(xs, af, words, wfc_t, bfc, wout_t, bout, wint_t, bint):
    def body(xs_ref, af_ref, w_ref, wfc_ref, bfc_ref, wout_ref, bout_ref,
             wint_ref, bint_ref, o_ref):
        s = jnp.sum(xs_ref[...] + af_ref[...], axis=0, keepdims=True) * (1.0 / N)
        m = jnp.dot(w_ref[...], wfc_ref[...], preferred_element_type=jnp.float32)
        m = m + bfc_ref[...]
        cat = jnp.concatenate([s, m], axis=1)
        for j in range(2):
            cat = jnp.dot(cat, wout_ref[j], preferred_element_type=jnp.float32)
            cat = jnp.maximum(cat + bout_ref[j], 0.0)
        o_ref[...] = (
            jnp.dot(cat, wint_ref[...], preferred_element_type=jnp.float32)
            + bint_ref[...]
        )

    return pl.pallas_call(
        body,
        out_shape=jax.ShapeDtypeStruct((1, 2), jnp.float32),
    )(xs, af, words.reshape(1, -1), wfc_t, bfc.reshape(1, -1), wout_t,
      bout.reshape(2, 1, 2 * DIM), wint_t, bint.reshape(1, -1))


def _sc_embed(table, idx):
    B = idx.shape[0]
    bpw = B // NW

    @functools.partial(
        pl.kernel,
        mesh=_mesh(),
        out_type=jax.ShapeDtypeStruct((B, DIM), jnp.float32),
        scratch_types=[
            pltpu.VMEM((bpw,), jnp.int32),
            pltpu.VMEM((bpw, DIM), jnp.float32),
            pltpu.SemaphoreType.DMA,
        ],
    )
    def k(table_h, idx_h, out_h, idx_v, rows_v, sem):
        base = _wid() * bpw
        pltpu.sync_copy(idx_h.at[pl.ds(base, bpw)], idx_v)
        cps = []
        for c in range(bpw // 64):
            sl = pl.ds(c * 64, 64)
            cps.append(pltpu.async_copy(table_h.at[idx_v.at[sl]],
                                        rows_v.at[sl, :], sem))
        for cp in cps:
            cp.wait()
        pltpu.sync_copy(rows_v, out_h.at[pl.ds(base, bpw), :])

    return k(table, idx)


def _sigmoid(x):
    return 1.0 / (1.0 + jnp.exp(-x))


def _sc_nf(aw, bw, adl_flat, bdl_flat, af, bias):
    rpw = NPAD // NW
    C = 16
    nch = rpw // C

    @functools.partial(
        pl.kernel,
        mesh=_mesh(),
        out_type=jax.ShapeDtypeStruct((NPAD, DIM), jnp.float32),
        scratch_types=[
            pltpu.VMEM((C * DEG,), jnp.int32),
            pltpu.VMEM((C * DEG,), jnp.int32),
            pltpu.VMEM((C * DEG, DIM), jnp.float32),
            pltpu.VMEM((C * DEG, DIM), jnp.float32),
            pltpu.VMEM((C, DIM), jnp.float32),
            pltpu.VMEM((C, DIM), jnp.float32),
            pltpu.VMEM((DIM,), jnp.float32),
            pltpu.SemaphoreType.DMA,
        ],
    )
    def k(aw_h, bw_h, adl_h, bdl_h, af_h, bias_h, out_h,
          ia_v, ib_v, ar_v, br_v, af_v, o_v, b_v, sem):
        wid = _wid()
        pltpu.sync_copy(bias_h, b_v)
        bias_vecs = [b_v[pl.ds(v * 16, 16)] for v in range(8)]

        def chunk(c, _):
            base = wid * rpw + c * C
            pltpu.sync_copy(adl_h.at[pl.ds(base * DEG, C * DEG)], ia_v)
            pltpu.sync_copy(bdl_h.at[pl.ds(base * DEG, C * DEG)], ib_v)
            cps = []
            for h in range(C * DEG // 128):
                sl = pl.ds(h * 128, 128)
                cps.append(pltpu.async_copy(aw_h.at[ia_v.at[sl]],
                                            ar_v.at[sl, :], sem))
                cps.append(pltpu.async_copy(bw_h.at[ib_v.at[sl]],
                                            br_v.at[sl, :], sem))
            pltpu.sync_copy(af_h.at[pl.ds(base, C), :], af_v)
            for cp in cps:
                cp.wait()

            def atom(a, _):
                r0 = a * DEG

                def dbody(d, accs):
                    r = r0 + d
                    out = []
                    for v in range(8):
                        sl = pl.ds(v * 16, 16)
                        x = ar_v[r, sl] + br_v[r, sl] + bias_vecs[v]
                        out.append(accs[v] + jnp.maximum(x, 0.0)
                                   + 0.01 * jnp.minimum(x, 0.0))
                    return tuple(out)

                accs = lax.fori_loop(
                    0, DEG, dbody,
                    tuple(jnp.zeros((16,), jnp.float32) for _ in range(8)))
                for v in range(8):
                    sl = pl.ds(v * 16, 16)
                    o_v[a, sl] = _sigmoid(af_v[a, sl] + accs[v])
                return 0

            lax.fori_loop(0, C, atom, 0)
            pltpu.sync_copy(o_v, out_h.at[pl.ds(base, C), :])
            return 0

        lax.fori_loop(0, nch, chunk, 0)

    return k(aw, bw, adl_flat, bdl_flat, af, bias)


def _sc_bond(bf, sw, i0, i1, bias):
    rpw = NB // NW
    E = 40
    nch = rpw // E

    @functools.partial(
        pl.kernel,
        mesh=_mesh(),
        out_type=jax.ShapeDtypeStruct((NB, DIM), jnp.float32),
        scratch_types=[
            pltpu.VMEM((E,), jnp.int32),
            pltpu.VMEM((E,), jnp.int32),
            pltpu.VMEM((E, DIM), jnp.float32),
            pltpu.VMEM((E, DIM), jnp.float32),
            pltpu.VMEM((E, DIM), jnp.float32),
            pltpu.VMEM((E, DIM), jnp.float32),
            pltpu.VMEM((DIM,), jnp.float32),
            pltpu.SemaphoreType.DMA,
        ],
    )
    def k(bf_h, sw_h, i0_h, i1_h, bias_h, out_h,
          i0_v, i1_v, bf_v, g0_v, g1_v, o_v, b_v, sem):
        wid = _wid()
        pltpu.sync_copy(bias_h, b_v)
        bias_vecs = [b_v[pl.ds(v * 16, 16)] for v in range(8)]

        def chunk(c, _):
            base = wid * rpw + c * E
            pltpu.sync_copy(i0_h.at[pl.ds(base, E)], i0_v)
            pltpu.sync_copy(i1_h.at[pl.ds(base, E)], i1_v)
            cp0 = pltpu.async_copy(sw_h.at[i0_v], g0_v, sem)
            cp1 = pltpu.async_copy(sw_h.at[i1_v], g1_v, sem)
            pltpu.sync_copy(bf_h.at[pl.ds(base, E), :], bf_v)
            cp0.wait()
            cp1.wait()

            def row(e, _):
                for v in range(8):
                    sl = pl.ds(v * 16, 16)
                    x = bf_v[e, sl] + g0_v[e, sl] + g1_v[e, sl] + bias_vecs[v]
                    o_v[e, sl] = _sigmoid(x)
                return 0

            lax.fori_loop(0, E, row, 0)
            pltpu.sync_copy(o_v, out_h.at[pl.ds(base, E), :])
            return 0

        lax.fori_loop(0, nch, chunk, 0)

    return k(bf, sw, i0, i1, bias)


def kernel(fingerprints, atom_degree_list, bond_feature, bond_degree_list,
           i_bond_j, adjacency, words, embed_table, W_bond, b_bond, W_nfc,
           b_nfc, W_sfc, b_sfc, W_sub, b_sub, W_fc, b_fc, W_out, b_out,
           W_int, b_int):
    fp = jnp.pad(fingerprints.astype(jnp.int32), (0, NPAD - N))
    xs = _sc_embed(embed_table, fp)[:N]

    for i in range(2):
        hs = _linear(xs, W_sub[i].T, b_sub[i], "relu", R=1000)
        xs = _adj_step(adjacency, hs, xs)

    bf = _linear(bond_feature, W_bond.T, b_bond, None, R=1600)
    af_p = jnp.pad(xs, ((0, NPAD - N), (0, 0)))
    adlf = jnp.pad(atom_degree_list.astype(jnp.int32).reshape(-1),
                   (0, (NPAD - N) * DEG))
    bdlf = jnp.pad(bond_degree_list.astype(jnp.int32).reshape(-1),
                   (0, (NPAD - N) * DEG))
    i0 = i_bond_j[:, 0].astype(jnp.int32)
    i1 = i_bond_j[:, 1].astype(jnp.int32)

    for i in range(3):
        wa_t = W_nfc[i, :, :DIM].T
        wb_t = W_nfc[i, :, DIM:].T
        aw = _linear(af_p, wa_t, None, None, R=1024)
        bw = _linear(bf, wb_t, None, None, R=1600)
        af_p = _sc_nf(aw, bw, adlf, bdlf, af_p, b_nfc[i])
        if i < 2:
            sw = _linear(af_p, W_sfc[i].T, None, None, R=1024)
            bf = _sc_bond(bf, sw, i0, i1, b_sfc[i])

    return _epilogue(xs, af_p[:N], words, W_fc.T, b_fc,
                     jnp.transpose(W_out, (0, 2, 1)), b_out, W_int.T, b_int)
